# baseline (speedup 1.0000x reference)
import functools

import jax
import jax.numpy as jnp
from jax import lax
from jax.experimental import pallas as pl
from jax.experimental.pallas import tpu as pltpu
from jax.experimental.pallas import tpu_sc as plsc

B, L, H = 4096, 50, 128
NC, NS = 2, 16
NW = NC * NS
TOK = B * L
TPW = TOK // NW
CH = 40
NCHUNK = TPW // CH
NREL, NTYP = 17, 4
NRT = NREL * NTYP


def _sc_body(it_hbm, io_hbm, irt_hbm, wtab, otab, rtab, xtab,
             emb_out, ie_out,
             itv, iov, irtv, wb0, wb1, wb2, wb3, ac0, ac1, ac2, ac3,
             rtb0, rtb1, rtb2, rtb3,
             rt_hbm, g0, g1, g2, g3, w0, w1, w2, w3):
    c = lax.axis_index("c")
    s = lax.axis_index("s")
    wid = s * NC + c

    @pl.when(s == 0)
    def _():
        pltpu.sync_copy(rtab, wb0.at[pl.ds(0, NREL)])
        pltpu.sync_copy(xtab, wb0.at[pl.ds(NREL, NTYP)])

        def fuse(r, carry):
            for t in range(NTYP):
                for cc in range(H // 16):
                    sl = pl.ds(cc * 16, 16)
                    ac0[r * NTYP + t, sl] = wb0[r, sl] + wb0[NREL + t, sl]
            return carry

        lax.fori_loop(0, NREL, fuse, 0)
        pltpu.sync_copy(ac0.at[pl.ds(0, NRT)], rt_hbm)

    pltpu.sync_copy(it_hbm.at[wid], itv)
    pltpu.sync_copy(io_hbm.at[wid], iov)
    pltpu.sync_copy(irt_hbm.at[wid], irtv)
    plsc.subcore_barrier()

    wbs, accs, rtbs = ((wb0, wb1, wb2, wb3), (ac0, ac1, ac2, ac3),
                       (rtb0, rtb1, rtb2, rtb3))
    gs, ws = (g0, g1, g2, g3), (w0, w1, w2, w3)

    def fire_g(j, b):
        pltpu.async_copy(wtab.at[itv.at[j]], wbs[b], gs[b])
        pltpu.async_copy(otab.at[iov.at[j]], accs[b], gs[b])
        pltpu.async_copy(rt_hbm.at[irtv.at[j]], rtbs[b], gs[b])

    def wait_g(j, b):
        pltpu.make_async_copy(wtab.at[itv.at[j]], wbs[b], gs[b]).wait()
        pltpu.make_async_copy(otab.at[iov.at[j]], accs[b], gs[b]).wait()
        pltpu.make_async_copy(rt_hbm.at[irtv.at[j]], rtbs[b], gs[b]).wait()

    def fire_w(j, b):
        base = wid * TPW + j * CH
        pltpu.async_copy(wbs[b], ie_out.at[pl.ds(base, CH)], ws[b])
        pltpu.async_copy(accs[b], emb_out.at[pl.ds(base, CH)], ws[b])

    def wait_w(j, b):
        base = wid * TPW + j * CH
        pltpu.make_async_copy(wbs[b], ie_out.at[pl.ds(base, CH)], ws[b]).wait()
        pltpu.make_async_copy(accs[b], emb_out.at[pl.ds(base, CH)], ws[b]).wait()

    def compute(j, b):
        wb, acc, rtb = wbs[b], accs[b], rtbs[b]

        @plsc.parallel_loop(0, CH, unroll=4)
        def _pass(t):
            for cc in range(H // 16):
                sl = pl.ds(cc * 16, 16)
                plsc.addupdate(acc.at[t, sl], wb[t, sl] + rtb[t, sl])

    def step(j, b):
        nb = (b + 2) % 4

        @pl.when(j + 2 < NCHUNK)
        def _():
            @pl.when(j >= 2)
            def _():
                wait_w(j - 2, nb)

            fire_g(j + 2, nb)

        wait_g(j, b)
        compute(j, b)
        fire_w(j, b)

    fire_g(0, 0)
    fire_g(1, 1)

    def quad(p, carry):
        for q in range(4):
            step(4 * p + q, q)
        return carry

    lax.fori_loop(0, NCHUNK // 4, quad, 0)
    for jj in range(NCHUNK - 4, NCHUNK):
        wait_w(jj, jj % 4)


@functools.partial(
    pl.kernel,
    out_type=(jax.ShapeDtypeStruct((TOK, H), jnp.float32),
              jax.ShapeDtypeStruct((TOK, H), jnp.float32)),
    mesh=plsc.VectorSubcoreMesh(core_axis_name="c", subcore_axis_name="s"),
    scratch_types=(
        pltpu.VMEM((NCHUNK, CH), jnp.int32),
        pltpu.VMEM((NCHUNK, CH), jnp.int32),
        pltpu.VMEM((NCHUNK, CH), jnp.int32),
        pltpu.VMEM((CH, H), jnp.float32),
        pltpu.VMEM((CH, H), jnp.float32),
        pltpu.VMEM((CH, H), jnp.float32),
        pltpu.VMEM((CH, H), jnp.float32),
        pltpu.VMEM((CH, H), jnp.float32),
        pltpu.VMEM((CH, H), jnp.float32),
        pltpu.VMEM((CH, H), jnp.float32),
        pltpu.VMEM((CH, H), jnp.float32),
        pltpu.VMEM((CH, H), jnp.float32),
        pltpu.VMEM((CH, H), jnp.float32),
        pltpu.VMEM((CH, H), jnp.float32),
        pltpu.VMEM((CH, H), jnp.float32),
        pltpu.HBM((NRT, H), jnp.float32),
        pltpu.SemaphoreType.DMA,
        pltpu.SemaphoreType.DMA,
        pltpu.SemaphoreType.DMA,
        pltpu.SemaphoreType.DMA,
        pltpu.SemaphoreType.DMA,
        pltpu.SemaphoreType.DMA,
        pltpu.SemaphoreType.DMA,
        pltpu.SemaphoreType.DMA,
    ),
)
def _sc_embed(*args):
    _sc_body(*args)


def kernel(input_token, input_obj_id, segment_label, token_type,
           word_emb, obj_emb, rel_emb, type_emb):
    it = input_token.T.reshape(NW, NCHUNK, CH).astype(jnp.int32)
    io = input_obj_id.T.reshape(NW, NCHUNK, CH).astype(jnp.int32)
    irt = (segment_label.astype(jnp.int32) * NTYP
           + token_type.astype(jnp.int32)).T.reshape(NW, NCHUNK, CH)
    emb, ie = _sc_embed(it, io, irt, word_emb, obj_emb, rel_emb, type_emb)
    emb = emb.reshape(L, B, H).transpose(1, 0, 2)
    ie = ie.reshape(L, B, H).transpose(1, 0, 2)
    return emb, ie


# speedup vs baseline: 1.0216x; 1.0216x over previous
import functools

import jax
import jax.numpy as jnp
from jax import lax
from jax.experimental import pallas as pl
from jax.experimental.pallas import tpu as pltpu
from jax.experimental.pallas import tpu_sc as plsc

B, L, H = 4096, 50, 128
NC, NS = 2, 16
NW = NC * NS
TOK = B * L
TPW = TOK // NW
CH = 80
NCHUNK = TPW // CH
NREL, NTYP = 17, 4
NRT = NREL * NTYP


def _sc_body(it_hbm, io_hbm, irt_hbm, wtab, otab, rtab, xtab,
             emb_out, ie_out,
             itv, iov, irtv, wb0, wb1, wb2, ac0, ac1, ac2, rtb0, rtb1, rtb2,
             rt_hbm, g0, g1, g2, w0, w1, w2):
    c = lax.axis_index("c")
    s = lax.axis_index("s")
    wid = s * NC + c

    @pl.when(s == 0)
    def _():
        pltpu.sync_copy(rtab, wb0.at[pl.ds(0, NREL)])
        pltpu.sync_copy(xtab, wb0.at[pl.ds(NREL, NTYP)])

        def fuse(r, carry):
            for t in range(NTYP):
                for cc in range(H // 16):
                    sl = pl.ds(cc * 16, 16)
                    ac0[r * NTYP + t, sl] = wb0[r, sl] + wb0[NREL + t, sl]
            return carry

        lax.fori_loop(0, NREL, fuse, 0)
        pltpu.sync_copy(ac0.at[pl.ds(0, NRT)], rt_hbm)

    pltpu.sync_copy(it_hbm.at[wid], itv)
    pltpu.sync_copy(io_hbm.at[wid], iov)
    pltpu.sync_copy(irt_hbm.at[wid], irtv)
    plsc.subcore_barrier()

    wbs, accs, rtbs = (wb0, wb1, wb2), (ac0, ac1, ac2), (rtb0, rtb1, rtb2)
    gs, ws = (g0, g1, g2), (w0, w1, w2)

    def fire_g(j, b):
        pltpu.async_copy(wtab.at[itv.at[j]], wbs[b], gs[b])
        pltpu.async_copy(otab.at[iov.at[j]], accs[b], gs[b])
        pltpu.async_copy(rt_hbm.at[irtv.at[j]], rtbs[b], gs[b])

    def wait_g(j, b):
        pltpu.make_async_copy(wtab.at[itv.at[j]], wbs[b], gs[b]).wait()
        pltpu.make_async_copy(otab.at[iov.at[j]], accs[b], gs[b]).wait()
        pltpu.make_async_copy(rt_hbm.at[irtv.at[j]], rtbs[b], gs[b]).wait()

    def fire_w(j, b):
        base = wid * TPW + j * CH
        pltpu.async_copy(wbs[b], ie_out.at[pl.ds(base, CH)], ws[b])
        pltpu.async_copy(accs[b], emb_out.at[pl.ds(base, CH)], ws[b])

    def wait_w(j, b):
        base = wid * TPW + j * CH
        pltpu.make_async_copy(wbs[b], ie_out.at[pl.ds(base, CH)], ws[b]).wait()
        pltpu.make_async_copy(accs[b], emb_out.at[pl.ds(base, CH)], ws[b]).wait()

    def compute(j, b):
        wb, acc, rtb = wbs[b], accs[b], rtbs[b]

        @plsc.parallel_loop(0, CH, unroll=4)
        def _pass(t):
            for cc in range(H // 16):
                sl = pl.ds(cc * 16, 16)
                plsc.addupdate(acc.at[t, sl], wb[t, sl] + rtb[t, sl])

    def step(j, b):
        nb = (b + 1) % 3

        @pl.when(j + 1 < NCHUNK)
        def _():
            @pl.when(j >= 2)
            def _():
                wait_w(j - 2, nb)

            fire_g(j + 1, nb)

        wait_g(j, b)
        compute(j, b)
        fire_w(j, b)

    fire_g(0, 0)

    def trio(p, carry):
        for q in range(3):
            step(3 * p + q, q)
        return carry

    lax.fori_loop(0, NCHUNK // 3, trio, 0)
    for j in range(NCHUNK - (NCHUNK % 3), NCHUNK):
        step(j, j % 3)
    wait_w(NCHUNK - 3, (NCHUNK - 3) % 3)
    wait_w(NCHUNK - 2, (NCHUNK - 2) % 3)
    wait_w(NCHUNK - 1, (NCHUNK - 1) % 3)


@functools.partial(
    pl.kernel,
    out_type=(jax.ShapeDtypeStruct((TOK, H), jnp.float32),
              jax.ShapeDtypeStruct((TOK, H), jnp.float32)),
    mesh=plsc.VectorSubcoreMesh(core_axis_name="c", subcore_axis_name="s"),
    scratch_types=(
        pltpu.VMEM((NCHUNK, CH), jnp.int32),
        pltpu.VMEM((NCHUNK, CH), jnp.int32),
        pltpu.VMEM((NCHUNK, CH), jnp.int32),
        pltpu.VMEM((CH, H), jnp.float32),
        pltpu.VMEM((CH, H), jnp.float32),
        pltpu.VMEM((CH, H), jnp.float32),
        pltpu.VMEM((CH, H), jnp.float32),
        pltpu.VMEM((CH, H), jnp.float32),
        pltpu.VMEM((CH, H), jnp.float32),
        pltpu.VMEM((CH, H), jnp.float32),
        pltpu.VMEM((CH, H), jnp.float32),
        pltpu.VMEM((CH, H), jnp.float32),
        pltpu.HBM((NRT, H), jnp.float32),
        pltpu.SemaphoreType.DMA,
        pltpu.SemaphoreType.DMA,
        pltpu.SemaphoreType.DMA,
        pltpu.SemaphoreType.DMA,
        pltpu.SemaphoreType.DMA,
        pltpu.SemaphoreType.DMA,
    ),
)
def _sc_embed(*args):
    _sc_body(*args)


def kernel(input_token, input_obj_id, segment_label, token_type,
           word_emb, obj_emb, rel_emb, type_emb):
    it = input_token.T.reshape(NW, NCHUNK, CH).astype(jnp.int32)
    io = input_obj_id.T.reshape(NW, NCHUNK, CH).astype(jnp.int32)
    irt = (segment_label.astype(jnp.int32) * NTYP
           + token_type.astype(jnp.int32)).T.reshape(NW, NCHUNK, CH)
    emb, ie = _sc_embed(it, io, irt, word_emb, obj_emb, rel_emb, type_emb)
    emb = emb.reshape(L, B, H).transpose(1, 0, 2)
    ie = ie.reshape(L, B, H).transpose(1, 0, 2)
    return emb, ie
